# baseline (speedup 1.0000x reference)
import functools

import jax
import jax.numpy as jnp
from jax import lax
from jax.experimental import pallas as pl
from jax.experimental.pallas import tpu as pltpu
from jax.experimental.pallas import tpu_sc as plsc

_N = 10000
_E = 320000
_D = 128
_H = 64
_W0 = _H + 16
_NPAD = 10240
_SACR = _N
_NSC = 2
_NTILE = 16
_GPT = 160
_EPAD = _NTILE * _GPT * 128
_KMAC = 8
_NMAC = _GPT // _KMAC
_ROWS_PT = _NPAD // _NTILE
_EPS = 1e-5


def _sc_agg(ysplit, src2d, dst2d, width2):
    mesh = plsc.VectorSubcoreMesh(core_axis_name="c", subcore_axis_name="s")

    @functools.partial(
        pl.kernel,
        mesh=mesh,
        out_type=jax.ShapeDtypeStruct((_NSC, _NPAD, width2), jnp.float32),
        compiler_params=pltpu.CompilerParams(use_tc_tiling_on_sc=False),
        scratch_types=[
            pltpu.VMEM((2, _KMAC, 128), jnp.int32),
            pltpu.VMEM((2, _KMAC, 128), jnp.int32),
            pltpu.VMEM((2, _KMAC * 128, width2), jnp.float32),
            pltpu.VMEM((64, width2), jnp.float32),
            pltpu.VMEM_SHARED((_NPAD, width2), jnp.float32),
            pltpu.SemaphoreType.DMA,
            pltpu.SemaphoreType.DMA,
        ],
    )
    def k(y_hbm, src_hbm, dst_hbm, out_hbm, sidx, didx, rows, zbuf, acc,
          gsem0, gsem1):
        c = lax.axis_index("c")
        s = lax.axis_index("s")
        npair = _NMAC // 2

        def zrow(i, carry):
            for j in range(width2 // 16):
                zbuf[i, pl.ds(j * 16, 16)] = jnp.zeros((16,), jnp.float32)
            return carry

        lax.fori_loop(0, 64, zrow, 0)
        for t in range(_ROWS_PT // 64):
            pltpu.sync_copy(zbuf, acc.at[pl.ds(s * _ROWS_PT + t * 64, 64)])
        plsc.subcore_barrier()

        def load_idx(chunk, b):
            gb = s * _GPT + chunk * _KMAC
            pltpu.sync_copy(src_hbm.at[pl.ds(gb, _KMAC)], sidx.at[b])
            pltpu.sync_copy(dst_hbm.at[pl.ds(gb, _KMAC)], didx.at[b])

        def fire_gathers(b, sem):
            return [
                pltpu.async_copy(
                    y_hbm.at[c].at[sidx.at[b].at[j]],
                    rows.at[b].at[pl.ds(j * 128, 128)], sem)
                for j in range(_KMAC)
            ]

        def scatters(b):
            for j in range(_KMAC):
                pltpu.sync_copy(
                    rows.at[b].at[pl.ds(j * 128, 128)],
                    acc.at[didx.at[b].at[j]], add=True)

        load_idx(0, 0)
        for cp in fire_gathers(0, gsem0):
            pass

        def body(m, carry):
            load_idx(2 * m + 1, 1)
            for j in range(_KMAC):
                pltpu.make_async_copy(
                    y_hbm.at[c].at[sidx.at[0].at[j]],
                    rows.at[0].at[pl.ds(j * 128, 128)], gsem0).wait()
            g1 = fire_gathers(1, gsem1)
            scatters(0)

            @pl.when(m < npair - 1)
            def _():
                load_idx(2 * m + 2, 0)
            for cp in g1:
                cp.wait()

            @pl.when(m < npair - 1)
            def _():
                fire_gathers(0, gsem0)
            scatters(1)
            return carry

        lax.fori_loop(0, npair, body, 0)

        plsc.subcore_barrier()
        pltpu.sync_copy(acc.at[pl.ds(s * _ROWS_PT, _ROWS_PT)],
                        rows.at[0].at[pl.ds(0, _ROWS_PT)])
        pltpu.sync_copy(rows.at[0].at[pl.ds(0, _ROWS_PT)],
                        out_hbm.at[c, pl.ds(s * _ROWS_PT, _ROWS_PT)])

    return k(ysplit, src2d, dst2d)


def _sc_agg_es(y, src2d, dst2d, width):
    gpt = _EPAD // 128 // (_NSC * _NTILE)
    kmac = 4
    nmac = gpt // kmac
    npair = nmac // 2
    mesh = plsc.VectorSubcoreMesh(core_axis_name="c", subcore_axis_name="s")

    @functools.partial(
        pl.kernel,
        mesh=mesh,
        out_type=jax.ShapeDtypeStruct((_NSC, _NPAD, width), jnp.float32),
        compiler_params=pltpu.CompilerParams(use_tc_tiling_on_sc=False),
        scratch_types=[
            pltpu.VMEM((2, kmac, 128), jnp.int32),
            pltpu.VMEM((2, kmac, 128), jnp.int32),
            pltpu.VMEM((2, kmac * 128, width), jnp.float32),
            pltpu.VMEM((64, width), jnp.float32),
            pltpu.VMEM_SHARED((_NPAD, width), jnp.float32),
            pltpu.SemaphoreType.DMA,
            pltpu.SemaphoreType.DMA,
        ],
    )
    def k(y_hbm, src_hbm, dst_hbm, out_hbm, sidx, didx, rows, zbuf, acc,
          gsem0, gsem1):
        c = lax.axis_index("c")
        s = lax.axis_index("s")
        base = (c * _NTILE + s) * gpt

        def zrow(i, carry):
            for j in range(width // 16):
                zbuf[i, pl.ds(j * 16, 16)] = jnp.zeros((16,), jnp.float32)
            return carry

        lax.fori_loop(0, 64, zrow, 0)
        for t in range(_ROWS_PT // 64):
            pltpu.sync_copy(zbuf, acc.at[pl.ds(s * _ROWS_PT + t * 64, 64)])
        plsc.subcore_barrier()

        def load_idx(chunk, b):
            gb = base + chunk * kmac
            pltpu.sync_copy(src_hbm.at[pl.ds(gb, kmac)], sidx.at[b])
            pltpu.sync_copy(dst_hbm.at[pl.ds(gb, kmac)], didx.at[b])

        def fire_gathers(b, sem):
            return [
                pltpu.async_copy(
                    y_hbm.at[sidx.at[b].at[j]],
                    rows.at[b].at[pl.ds(j * 128, 128)], sem)
                for j in range(kmac)
            ]

        def scatters(b):
            for j in range(kmac):
                pltpu.sync_copy(
                    rows.at[b].at[pl.ds(j * 128, 128)],
                    acc.at[didx.at[b].at[j]], add=True)

        load_idx(0, 0)
        for cp in fire_gathers(0, gsem0):
            pass

        def body(m, carry):
            load_idx(2 * m + 1, 1)
            for j in range(kmac):
                pltpu.make_async_copy(
                    y_hbm.at[sidx.at[0].at[j]],
                    rows.at[0].at[pl.ds(j * 128, 128)], gsem0).wait()
            g1 = fire_gathers(1, gsem1)
            scatters(0)

            @pl.when(m < npair - 1)
            def _():
                load_idx(2 * m + 2, 0)
            for cp in g1:
                cp.wait()

            @pl.when(m < npair - 1)
            def _():
                fire_gathers(0, gsem0)
            scatters(1)
            return carry

        lax.fori_loop(0, npair, body, 0)

        plsc.subcore_barrier()
        half = _ROWS_PT // 2
        for t in range(2):
            pltpu.sync_copy(acc.at[pl.ds(s * _ROWS_PT + t * half, half)],
                            rows.at[0].at[pl.ds(0, half)])
            pltpu.sync_copy(rows.at[0].at[pl.ds(0, half)],
                            out_hbm.at[c, pl.ds(s * _ROWS_PT + t * half, half)])

    return k(y, src2d, dst2d)


def _dot_t(a, w):
    return lax.dot_general(a, w, (((1,), (1,)), ((), ())),
                           precision=lax.Precision.HIGHEST,
                           preferred_element_type=jnp.float32)


def _bn_relu(t, g, b):
    mu = jnp.mean(t, axis=0, keepdims=True)
    d = t - mu
    var = jnp.mean(d * d, axis=0, keepdims=True)
    return jnp.maximum(d / jnp.sqrt(var + _EPS) * g + b, 0.0)


def _split2(y):
    w2 = y.shape[1] // 2
    return jnp.stack([y[:, :w2], y[:, w2:]])


def _tc_pre(x, wl, wr):
    def body(x_ref, wl_ref, wr_ref, y_ref, z_ref):
        xx = x_ref[...]
        y = _dot_t(xx, wl_ref[...])
        col = lax.broadcasted_iota(jnp.int32, (_N, _W0 - _H), 1)
        ext = jnp.where(col == 0, 1.0, 0.0).astype(jnp.float32)
        y_ref[...] = _split2(jnp.concatenate([y, ext], axis=1))
        z_ref[...] = _dot_t(xx, wr_ref[...])

    return pl.pallas_call(
        body,
        compiler_params=pltpu.CompilerParams(
            vmem_limit_bytes=100 * 1024 * 1024),
        out_shape=[jax.ShapeDtypeStruct((2, _N, _W0 // 2), jnp.float32),
                   jax.ShapeDtypeStruct((_N, _H), jnp.float32)],
    )(x, wl, wr)


def _tc_post0(p, z, bl, g, b, wl_n, wr_n):
    def body(p_ref, z_ref, bl_ref, g_ref, b_ref, wl_ref, wr_ref,
             y_ref, zn_ref, dinv_ref):
        S = jnp.concatenate([p_ref[0], p_ref[1]], axis=1)
        agg = S[:_N, :_H]
        dinv = 1.0 / jnp.maximum(S[:_N, _H:_H + 1], 1.0)
        t = agg * dinv + z_ref[...] + bl_ref[...]
        h = _bn_relu(t, g_ref[...], b_ref[...])
        y_ref[...] = _dot_t(h, wl_ref[...])
        zn_ref[...] = _dot_t(h, wr_ref[...])
        dinv_ref[...] = dinv

    return pl.pallas_call(
        body,
        compiler_params=pltpu.CompilerParams(
            vmem_limit_bytes=100 * 1024 * 1024),
        out_shape=[jax.ShapeDtypeStruct((_N, _H), jnp.float32),
                   jax.ShapeDtypeStruct((_N, _H), jnp.float32),
                   jax.ShapeDtypeStruct((_N, 1), jnp.float32)],
    )(p, z, bl, g, b, wl_n, wr_n)


def _tc_mid(p, dinv, z, bl, g, b, wl_n, wr_n):
    def body(p_ref, dinv_ref, z_ref, bl_ref, g_ref, b_ref, wl_ref, wr_ref,
             y_ref, zn_ref):
        S = p_ref[0] + p_ref[1]
        t = S[:_N, :] * dinv_ref[...] + z_ref[...] + bl_ref[...]
        h = _bn_relu(t, g_ref[...], b_ref[...])
        y_ref[...] = _dot_t(h, wl_ref[...])
        zn_ref[...] = _dot_t(h, wr_ref[...])

    return pl.pallas_call(
        body,
        compiler_params=pltpu.CompilerParams(
            vmem_limit_bytes=100 * 1024 * 1024),
        out_shape=[jax.ShapeDtypeStruct((_N, _H), jnp.float32),
                   jax.ShapeDtypeStruct((_N, _H), jnp.float32)],
    )(p, dinv, z, bl, g, b, wl_n, wr_n)


def _tc_fin(p, dinv, z, bl, g, b, wo, bo):
    def body(p_ref, dinv_ref, z_ref, bl_ref, g_ref, b_ref, wo_ref, bo_ref,
             o_ref):
        S = p_ref[0] + p_ref[1]
        t = S[:_N, :] * dinv_ref[...] + z_ref[...] + bl_ref[...]
        h = _bn_relu(t, g_ref[...], b_ref[...])
        o_ref[...] = _dot_t(h, wo_ref[...]) + bo_ref[0, 0]

    return pl.pallas_call(
        body,
        compiler_params=pltpu.CompilerParams(
            vmem_limit_bytes=100 * 1024 * 1024),
        out_shape=jax.ShapeDtypeStruct((_N, 8), jnp.float32),
    )(p, dinv, z, bl, g, b, wo, bo)


def kernel(x, edge_index, Wl0, bl0, Wr0, gamma0, beta0, Wl1, bl1, Wr1,
           gamma1, beta1, Wl2, bl2, Wr2, gamma2, beta2, Wout, bout):
    src = edge_index[0]
    dst = edge_index[1]
    pad = _EPAD - _E
    src2d = jnp.concatenate(
        [src, jnp.zeros((pad,), jnp.int32)]).reshape(-1, 128)
    pad_dst = _SACR + jnp.arange(pad, dtype=jnp.int32) % (_NPAD - _N)
    dst2d = jnp.concatenate([dst, pad_dst]).reshape(-1, 128)

    r = lambda v: v.reshape(1, -1)

    y0, z0 = _tc_pre(x, Wl0, Wr0)
    p0 = _sc_agg(y0, src2d, dst2d, _W0 // 2)
    y1, z1, dinv = _tc_post0(p0, z0, r(bl0), r(gamma0), r(beta0), Wl1, Wr1)
    p1 = _sc_agg_es(y1, src2d, dst2d, _H)
    y2, z2 = _tc_mid(p1, dinv, z1, r(bl1), r(gamma1), r(beta1), Wl2, Wr2)
    p2 = _sc_agg_es(y2, src2d, dst2d, _H)
    wo8 = jnp.concatenate([Wout, jnp.zeros((7, _H), jnp.float32)])
    out = _tc_fin(p2, dinv, z2, r(bl2), r(gamma2), r(beta2), wo8, r(bout))
    return out[:, 0]


# speedup vs baseline: 2.4029x; 2.4029x over previous
import functools

import jax
import jax.numpy as jnp
from jax import lax
from jax.experimental import pallas as pl
from jax.experimental.pallas import tpu as pltpu
from jax.experimental.pallas import tpu_sc as plsc

_N = 10000
_E = 320000
_D = 128
_H = 64
_W0 = _H + 16
_NPAD = 10240
_SACR = _N
_NSC = 2
_NTILE = 16
_GPT = 160
_EPAD = _NTILE * _GPT * 128
_KMAC = 8
_NMAC = _GPT // _KMAC
_ROWS_PT = _NPAD // _NTILE
_EPS = 1e-5


def _sc_agg(ysplit, src2d, dst2d, width2):
    mesh = plsc.VectorSubcoreMesh(core_axis_name="c", subcore_axis_name="s")

    @functools.partial(
        pl.kernel,
        mesh=mesh,
        out_type=jax.ShapeDtypeStruct((_NSC, _NPAD, width2), jnp.float32),
        compiler_params=pltpu.CompilerParams(use_tc_tiling_on_sc=False),
        scratch_types=[
            pltpu.VMEM((2, _KMAC, 128), jnp.int32),
            pltpu.VMEM((2, _KMAC, 128), jnp.int32),
            pltpu.VMEM((2, _KMAC * 128, width2), jnp.float32),
            pltpu.VMEM((64, width2), jnp.float32),
            pltpu.VMEM_SHARED((_NPAD, width2), jnp.float32),
            pltpu.SemaphoreType.DMA,
            pltpu.SemaphoreType.DMA,
        ],
    )
    def k(y_hbm, src_hbm, dst_hbm, out_hbm, sidx, didx, rows, zbuf, acc,
          gsem0, gsem1):
        c = lax.axis_index("c")
        s = lax.axis_index("s")
        npair = _NMAC // 2

        def zrow(i, carry):
            for j in range(width2 // 16):
                zbuf[i, pl.ds(j * 16, 16)] = jnp.zeros((16,), jnp.float32)
            return carry

        lax.fori_loop(0, 64, zrow, 0)
        for t in range(_ROWS_PT // 64):
            pltpu.sync_copy(zbuf, acc.at[pl.ds(s * _ROWS_PT + t * 64, 64)])
        plsc.subcore_barrier()

        def load_idx(chunk, b):
            gb = s * _GPT + chunk * _KMAC
            pltpu.sync_copy(src_hbm.at[pl.ds(gb, _KMAC)], sidx.at[b])
            pltpu.sync_copy(dst_hbm.at[pl.ds(gb, _KMAC)], didx.at[b])

        def fire_gathers(b, sem):
            return [
                pltpu.async_copy(
                    y_hbm.at[c].at[sidx.at[b].at[j]],
                    rows.at[b].at[pl.ds(j * 128, 128)], sem)
                for j in range(_KMAC)
            ]

        def scatters(b):
            for j in range(_KMAC):
                pltpu.sync_copy(
                    rows.at[b].at[pl.ds(j * 128, 128)],
                    acc.at[didx.at[b].at[j]], add=True)

        load_idx(0, 0)
        for cp in fire_gathers(0, gsem0):
            pass

        def body(m, carry):
            load_idx(2 * m + 1, 1)
            for j in range(_KMAC):
                pltpu.make_async_copy(
                    y_hbm.at[c].at[sidx.at[0].at[j]],
                    rows.at[0].at[pl.ds(j * 128, 128)], gsem0).wait()
            g1 = fire_gathers(1, gsem1)
            scatters(0)

            @pl.when(m < npair - 1)
            def _():
                load_idx(2 * m + 2, 0)
            for cp in g1:
                cp.wait()

            @pl.when(m < npair - 1)
            def _():
                fire_gathers(0, gsem0)
            scatters(1)
            return carry

        lax.fori_loop(0, npair, body, 0)

        plsc.subcore_barrier()
        pltpu.sync_copy(acc.at[pl.ds(s * _ROWS_PT, _ROWS_PT)],
                        rows.at[0].at[pl.ds(0, _ROWS_PT)])
        pltpu.sync_copy(rows.at[0].at[pl.ds(0, _ROWS_PT)],
                        out_hbm.at[c, pl.ds(s * _ROWS_PT, _ROWS_PT)])

    return k(ysplit, src2d, dst2d)


def _sc_agg_es(y, src2d, dst2d, width):
    gpt = _EPAD // 128 // (_NSC * _NTILE)
    kmac = 4
    nmac = gpt // kmac
    npair = nmac // 2
    mesh = plsc.VectorSubcoreMesh(core_axis_name="c", subcore_axis_name="s")

    @functools.partial(
        pl.kernel,
        mesh=mesh,
        out_type=jax.ShapeDtypeStruct((_NSC, _NPAD, width), jnp.float32),
        compiler_params=pltpu.CompilerParams(use_tc_tiling_on_sc=False),
        scratch_types=[
            pltpu.VMEM((2, kmac, 128), jnp.int32),
            pltpu.VMEM((2, kmac, 128), jnp.int32),
            pltpu.VMEM((2, kmac * 128, width), jnp.float32),
            pltpu.VMEM((64, width), jnp.float32),
            pltpu.VMEM_SHARED((_NPAD, width), jnp.float32),
            pltpu.SemaphoreType.DMA,
            pltpu.SemaphoreType.DMA,
        ],
    )
    def k(y_hbm, src_hbm, dst_hbm, out_hbm, sidx, didx, rows, zbuf, acc,
          gsem0, gsem1):
        c = lax.axis_index("c")
        s = lax.axis_index("s")
        base = (c * _NTILE + s) * gpt

        def zrow(i, carry):
            for j in range(width // 16):
                zbuf[i, pl.ds(j * 16, 16)] = jnp.zeros((16,), jnp.float32)
            return carry

        lax.fori_loop(0, 64, zrow, 0)
        for t in range(_ROWS_PT // 64):
            pltpu.sync_copy(zbuf, acc.at[pl.ds(s * _ROWS_PT + t * 64, 64)])
        plsc.subcore_barrier()

        def load_idx(chunk, b):
            gb = base + chunk * kmac
            pltpu.sync_copy(src_hbm.at[pl.ds(gb, kmac)], sidx.at[b])
            pltpu.sync_copy(dst_hbm.at[pl.ds(gb, kmac)], didx.at[b])

        def fire_gathers(b, sem):
            return [
                pltpu.async_copy(
                    y_hbm.at[sidx.at[b].at[j]],
                    rows.at[b].at[pl.ds(j * 128, 128)], sem)
                for j in range(kmac)
            ]

        def scatters(b):
            for j in range(kmac):
                pltpu.sync_copy(
                    rows.at[b].at[pl.ds(j * 128, 128)],
                    acc.at[didx.at[b].at[j]], add=True)

        load_idx(0, 0)
        for cp in fire_gathers(0, gsem0):
            pass

        def body(m, carry):
            load_idx(2 * m + 1, 1)
            for j in range(kmac):
                pltpu.make_async_copy(
                    y_hbm.at[sidx.at[0].at[j]],
                    rows.at[0].at[pl.ds(j * 128, 128)], gsem0).wait()
            g1 = fire_gathers(1, gsem1)
            scatters(0)

            @pl.when(m < npair - 1)
            def _():
                load_idx(2 * m + 2, 0)
            for cp in g1:
                cp.wait()

            @pl.when(m < npair - 1)
            def _():
                fire_gathers(0, gsem0)
            scatters(1)
            return carry

        lax.fori_loop(0, npair, body, 0)

        plsc.subcore_barrier()
        half = _ROWS_PT // 2
        for t in range(2):
            pltpu.sync_copy(acc.at[pl.ds(s * _ROWS_PT + t * half, half)],
                            rows.at[0].at[pl.ds(0, half)])
            pltpu.sync_copy(rows.at[0].at[pl.ds(0, half)],
                            out_hbm.at[c, pl.ds(s * _ROWS_PT + t * half, half)])

    return k(y, src2d, dst2d)


def _dot_t(a, w):
    return lax.dot_general(a, w, (((1,), (1,)), ((), ())),
                           precision=lax.Precision.HIGHEST,
                           preferred_element_type=jnp.float32)


def _bn_relu(t, g, b):
    mu = jnp.mean(t, axis=0, keepdims=True)
    d = t - mu
    var = jnp.mean(d * d, axis=0, keepdims=True)
    return jnp.maximum(d / jnp.sqrt(var + _EPS) * g + b, 0.0)


def _split2(y):
    w2 = y.shape[1] // 2
    return jnp.stack([y[:, :w2], y[:, w2:]])


def _tc_pre(x, wl, wr):
    def body(x_ref, wl_ref, wr_ref, y_ref, z_ref):
        xx = x_ref[...]
        y = _dot_t(xx, wl_ref[...])
        col = lax.broadcasted_iota(jnp.int32, (_N, _W0 - _H), 1)
        ext = jnp.where(col == 0, 1.0, 0.0).astype(jnp.float32)
        y_ref[...] = _split2(jnp.concatenate([y, ext], axis=1))
        z_ref[...] = _dot_t(xx, wr_ref[...])

    return pl.pallas_call(
        body,
        compiler_params=pltpu.CompilerParams(
            vmem_limit_bytes=100 * 1024 * 1024),
        out_shape=[jax.ShapeDtypeStruct((2, _N, _W0 // 2), jnp.float32),
                   jax.ShapeDtypeStruct((_N, _H), jnp.float32)],
    )(x, wl, wr)


def _tc_post0(p, z, bl, g, b, wl_n, wr_n):
    def body(p_ref, z_ref, bl_ref, g_ref, b_ref, wl_ref, wr_ref,
             y_ref, zn_ref, dinv_ref):
        S = jnp.concatenate([p_ref[0], p_ref[1]], axis=1)
        agg = S[:_N, :_H]
        dinv = 1.0 / jnp.maximum(S[:_N, _H:_H + 1], 1.0)
        t = agg * dinv + z_ref[...] + bl_ref[...]
        h = _bn_relu(t, g_ref[...], b_ref[...])
        y_ref[...] = _dot_t(h, wl_ref[...])
        zn_ref[...] = _dot_t(h, wr_ref[...])
        dinv_ref[...] = dinv

    return pl.pallas_call(
        body,
        compiler_params=pltpu.CompilerParams(
            vmem_limit_bytes=100 * 1024 * 1024),
        out_shape=[jax.ShapeDtypeStruct((_N, _H), jnp.float32),
                   jax.ShapeDtypeStruct((_N, _H), jnp.float32),
                   jax.ShapeDtypeStruct((_N, 1), jnp.float32)],
    )(p, z, bl, g, b, wl_n, wr_n)


def _tc_mid(p, dinv, z, bl, g, b, wl_n, wr_n):
    def body(p_ref, dinv_ref, z_ref, bl_ref, g_ref, b_ref, wl_ref, wr_ref,
             y_ref, zn_ref):
        S = p_ref[0] + p_ref[1]
        t = S[:_N, :] * dinv_ref[...] + z_ref[...] + bl_ref[...]
        h = _bn_relu(t, g_ref[...], b_ref[...])
        y_ref[...] = _dot_t(h, wl_ref[...])
        zn_ref[...] = _dot_t(h, wr_ref[...])

    return pl.pallas_call(
        body,
        compiler_params=pltpu.CompilerParams(
            vmem_limit_bytes=100 * 1024 * 1024),
        out_shape=[jax.ShapeDtypeStruct((_N, _H), jnp.float32),
                   jax.ShapeDtypeStruct((_N, _H), jnp.float32)],
    )(p, dinv, z, bl, g, b, wl_n, wr_n)


def _tc_fin(p, dinv, z, bl, g, b, wo, bo):
    def body(p_ref, dinv_ref, z_ref, bl_ref, g_ref, b_ref, wo_ref, bo_ref,
             o_ref):
        S = p_ref[0] + p_ref[1]
        t = S[:_N, :] * dinv_ref[...] + z_ref[...] + bl_ref[...]
        h = _bn_relu(t, g_ref[...], b_ref[...])
        o_ref[...] = _dot_t(h, wo_ref[...]) + bo_ref[0, 0]

    return pl.pallas_call(
        body,
        compiler_params=pltpu.CompilerParams(
            vmem_limit_bytes=100 * 1024 * 1024),
        out_shape=jax.ShapeDtypeStruct((_N, 8), jnp.float32),
    )(p, dinv, z, bl, g, b, wo, bo)


def kernel(x, edge_index, Wl0, bl0, Wr0, gamma0, beta0, Wl1, bl1, Wr1,
           gamma1, beta1, Wl2, bl2, Wr2, gamma2, beta2, Wout, bout):
    src = edge_index[0]
    dst = edge_index[1]
    pad = _EPAD - _E
    pad_src = jnp.arange(pad, dtype=jnp.int32) % _N
    src2d = jnp.concatenate([src, pad_src]).reshape(-1, 128)
    pad_dst = _SACR + jnp.arange(pad, dtype=jnp.int32) % (_NPAD - _N)
    dst2d = jnp.concatenate([dst, pad_dst]).reshape(-1, 128)

    r = lambda v: v.reshape(1, -1)

    y0, z0 = _tc_pre(x, Wl0, Wr0)
    p0 = _sc_agg(y0, src2d, dst2d, _W0 // 2)
    y1, z1, dinv = _tc_post0(p0, z0, r(bl0), r(gamma0), r(beta0), Wl1, Wr1)
    p1 = _sc_agg_es(y1, src2d, dst2d, _H)
    y2, z2 = _tc_mid(p1, dinv, z1, r(bl1), r(gamma1), r(beta1), Wl2, Wr2)
    p2 = _sc_agg_es(y2, src2d, dst2d, _H)
    wo8 = jnp.concatenate([Wout, jnp.zeros((7, _H), jnp.float32)])
    out = _tc_fin(p2, dinv, z2, r(bl2), r(gamma2), r(beta2), wo8, r(bout))
    return out[:, 0]
